# baseline (speedup 1.0000x reference)
import functools

import jax
import jax.numpy as jnp
from jax import lax
from jax.experimental import pallas as pl
from jax.experimental.pallas import tpu as pltpu
from jax.experimental.pallas import tpu_sc as plsc

_B = 16384
_D = 16
_CHUNK = 128


def _sc_gather(u_idx, i_idx, u_table, i_table):
    info = plsc.get_sparse_core_info()
    nw = info.num_cores * info.num_subcores
    b_per_w = _B // nw
    n_chunks = b_per_w // _CHUNK

    u_idx_r = u_idx.reshape(nw, n_chunks, _CHUNK)
    i_idx_r = i_idx.reshape(nw, n_chunks, _CHUNK)

    mesh = plsc.VectorSubcoreMesh(core_axis_name="c", subcore_axis_name="s")
    out_sds = jax.ShapeDtypeStruct((nw, n_chunks, _CHUNK, _D), jnp.float32)

    @functools.partial(
        pl.kernel,
        mesh=mesh,
        out_type=(out_sds, out_sds),
        compiler_params=pltpu.CompilerParams(use_tc_tiling_on_sc=False),
        scratch_types=[
            pltpu.VMEM((n_chunks, _CHUNK), jnp.int32),
            pltpu.VMEM((n_chunks, _CHUNK), jnp.int32),
            pltpu.VMEM((n_chunks, _CHUNK, _D), jnp.float32),
            pltpu.VMEM((n_chunks, _CHUNK, _D), jnp.float32),
            pltpu.SemaphoreType.DMA,
        ],
    )
    def gather_kernel(u_table_hbm, i_table_hbm, u_idx_hbm, i_idx_hbm,
                      u_out, i_out, uidx_v, iidx_v, urows_v, irows_v, sem):
        wid = lax.axis_index("s") * info.num_cores + lax.axis_index("c")
        pltpu.sync_copy(u_idx_hbm.at[wid], uidx_v)
        pltpu.sync_copy(i_idx_hbm.at[wid], iidx_v)
        copies = []
        for j in range(n_chunks):
            copies.append(
                pltpu.async_copy(u_table_hbm.at[uidx_v.at[j]], urows_v.at[j], sem))
            copies.append(
                pltpu.async_copy(i_table_hbm.at[iidx_v.at[j]], irows_v.at[j], sem))
        for c in copies:
            c.wait()
        pltpu.sync_copy(urows_v, u_out.at[wid])
        pltpu.sync_copy(irows_v, i_out.at[wid])

    u_rows, i_rows = gather_kernel(u_table, i_table, u_idx_r, i_idx_r)
    return u_rows.reshape(_B, _D), i_rows.reshape(_B, _D)


def _mlp_body(u_ref, i_ref, w1u_ref, w1i_ref, b1_ref, w2_ref, b2_ref,
              w3_ref, b3_ref, o_ref):
    h = (jnp.dot(u_ref[...], w1u_ref[...], preferred_element_type=jnp.float32)
         + jnp.dot(i_ref[...], w1i_ref[...], preferred_element_type=jnp.float32)
         + b1_ref[...])
    h = jnp.maximum(h, 0.0)
    h = jnp.dot(h, w2_ref[...], preferred_element_type=jnp.float32) + b2_ref[...]
    h = jnp.maximum(h, 0.0)
    z = jnp.sum(h * w3_ref[...], axis=1, keepdims=True) + b3_ref[...]
    o_ref[...] = 1.0 / (1.0 + jnp.exp(-z))


def _tc_mlp(u_e, i_e, W1, b1, W2, b2, W3, b3):
    w1u = W1[:_D, :]
    w1i = W1[_D:, :]
    b1r = b1.reshape(1, -1)
    b2r = b2.reshape(1, -1)
    w3r = W3.reshape(1, -1)
    b3r = b3.reshape(1, 1)
    out = pl.pallas_call(
        _mlp_body,
        out_shape=jax.ShapeDtypeStruct((_B, 1), jnp.float32),
    )(u_e, i_e, w1u, w1i, b1r, W2, b2r, w3r, b3r)
    return out.reshape(-1)


def kernel(u_idx, i_idx, u_table, i_table, W1, b1, W2, b2, W3, b3):
    u_e, i_e = _sc_gather(u_idx, i_idx, u_table, i_table)
    return _tc_mlp(u_e, i_e, W1, b1, W2, b2, W3, b3)
